# Initial kernel scaffold; baseline (speedup 1.0000x reference)
import jax
import jax.numpy as jnp
from jax.experimental import pallas as pl


def kernel(tokens, vals):
    raise NotImplementedError("write your pallas kernel here")


# speedup vs baseline: 3.5371x; 3.5371x over previous
import jax
import jax.numpy as jnp
from jax.experimental import pallas as pl

_N_TYPES = 100000
_SEQ_LEN = 200
_DBLK = 2048


def _bow_block_kernel(tok_ref, val_ref, out_ref):
    j = pl.program_id(0)
    col = jax.lax.broadcasted_iota(jnp.int32, (_SEQ_LEN, _DBLK), 1) + j * _DBLK
    mask = col == tok_ref[:, :]
    out_ref[:, 0, :] = jnp.where(mask, val_ref[:, :], 0.0)


def kernel(tokens, vals):
    tok2 = tokens.astype(jnp.int32).reshape(_SEQ_LEN, 1)
    val2 = vals.reshape(_SEQ_LEN, 1)
    grid = (pl.cdiv(_N_TYPES, _DBLK),)
    out = pl.pallas_call(
        _bow_block_kernel,
        grid=grid,
        in_specs=[
            pl.BlockSpec((_SEQ_LEN, 1), lambda j: (0, 0)),
            pl.BlockSpec((_SEQ_LEN, 1), lambda j: (0, 0)),
        ],
        out_specs=pl.BlockSpec((_SEQ_LEN, 1, _DBLK), lambda j: (0, 0, j)),
        out_shape=jax.ShapeDtypeStruct((_SEQ_LEN, 1, _N_TYPES), jnp.float32),
    )(tok2, val2)
    return out


# speedup vs baseline: 3.6069x; 1.0198x over previous
import jax
import jax.numpy as jnp
from jax.experimental import pallas as pl

_N_TYPES = 100000
_SEQ_LEN = 200
_DBLK = 2048


def _bow_block_kernel(tok_ref, val_ref, col_ref, out_ref):
    j = pl.program_id(0)
    tokrel = tok_ref[:, :] - j * _DBLK
    out_ref[:, :] = jnp.where(col_ref[:, :] == tokrel, val_ref[:, :], 0.0)


def kernel(tokens, vals):
    tok2 = jnp.broadcast_to(tokens.astype(jnp.int32)[:, None, None], (_SEQ_LEN, 1, _DBLK))
    val2 = jnp.broadcast_to(vals[:, None, None], (_SEQ_LEN, 1, _DBLK))
    col2 = jnp.broadcast_to(
        jnp.arange(_DBLK, dtype=jnp.int32)[None, None, :], (_SEQ_LEN, 1, _DBLK)
    )
    grid = (pl.cdiv(_N_TYPES, _DBLK),)
    out = pl.pallas_call(
        _bow_block_kernel,
        grid=grid,
        in_specs=[
            pl.BlockSpec((_SEQ_LEN, None, _DBLK), lambda j: (0, 0, 0)),
            pl.BlockSpec((_SEQ_LEN, None, _DBLK), lambda j: (0, 0, 0)),
            pl.BlockSpec((_SEQ_LEN, None, _DBLK), lambda j: (0, 0, 0)),
        ],
        out_specs=pl.BlockSpec((_SEQ_LEN, None, _DBLK), lambda j: (0, 0, j)),
        out_shape=jax.ShapeDtypeStruct((_SEQ_LEN, 1, _N_TYPES), jnp.float32),
    )(tok2, val2, col2)
    return out


# speedup vs baseline: 3.9006x; 1.0814x over previous
import jax
import jax.numpy as jnp
from jax.experimental import pallas as pl

_N_TYPES = 100000
_SEQ_LEN = 200
_DBLK = 2048


def _bow_block_kernel(tokcol_ref, val_ref, out_ref):
    j = pl.program_id(0)
    mask = tokcol_ref[:, :] == j * _DBLK
    out_ref[:, :] = jnp.where(mask, val_ref[:, :], 0.0)


def kernel(tokens, vals):
    tokcol = (
        tokens.astype(jnp.int32)[:, None, None]
        - jnp.arange(_DBLK, dtype=jnp.int32)[None, None, :]
    )
    val2 = jnp.broadcast_to(vals[:, None, None], (_SEQ_LEN, 1, _DBLK))
    grid = (pl.cdiv(_N_TYPES, _DBLK),)
    out = pl.pallas_call(
        _bow_block_kernel,
        grid=grid,
        in_specs=[
            pl.BlockSpec((_SEQ_LEN, None, _DBLK), lambda j: (0, 0, 0)),
            pl.BlockSpec((_SEQ_LEN, None, _DBLK), lambda j: (0, 0, 0)),
        ],
        out_specs=pl.BlockSpec((_SEQ_LEN, None, _DBLK), lambda j: (0, 0, j)),
        out_shape=jax.ShapeDtypeStruct((_SEQ_LEN, 1, _N_TYPES), jnp.float32),
    )(tokcol, val2)
    return out
